# baseline (speedup 1.0000x reference)
import functools

import jax
import jax.numpy as jnp
from jax import lax
from jax.experimental import pallas as pl
from jax.experimental.pallas import tpu as pltpu
from jax.experimental.pallas import tpu_sc as plsc

N = 10000
E = 320000
DF = 128
DH = 8
DE = 16

NC, NS = 2, 16
NW = NC * NS
EPT = E // NW
SUB = 125
NSUB = 8
CH = SUB * NSUB
NCHUNK = EPT // CH
RPT = N // NS
GSUB = E // SUB

_MESH = plsc.VectorSubcoreMesh(core_axis_name="c", subcore_axis_name="s")
_SC_PARAMS = pltpu.CompilerParams(needs_layout_passes=False,
                                  use_tc_tiling_on_sc=False)


def _tc1_body(x_ref, w1_ref, eat_ref, wet_ref, h0_ref, gatet_ref):
    h0_ref[...] = jnp.maximum(
        jnp.dot(x_ref[...], w1_ref[...], preferred_element_type=jnp.float32),
        0.0)
    g = jnp.dot(wet_ref[...], eat_ref[...], preferred_element_type=jnp.float32)
    gatet_ref[...] = 1.0 / (1.0 + jnp.exp(-g))


def _tc1(x, W1, ea, We):
    grid = 10
    nb = N // grid
    eb = E // grid
    return pl.pallas_call(
        _tc1_body,
        grid=(grid,),
        in_specs=[
            pl.BlockSpec((nb, DF), lambda i: (i, 0)),
            pl.BlockSpec((DF, DH), lambda i: (0, 0)),
            pl.BlockSpec((DE, eb), lambda i: (0, i)),
            pl.BlockSpec((DH, DE), lambda i: (0, 0)),
        ],
        out_specs=[
            pl.BlockSpec((nb, DH), lambda i: (i, 0)),
            pl.BlockSpec((DH, eb), lambda i: (0, i)),
        ],
        out_shape=[
            jax.ShapeDtypeStruct((N, DH), jnp.float32),
            jax.ShapeDtypeStruct((DH, E), jnp.float32),
        ],
    )(x, W1, ea.T, We.T)


def _sca_body(h0_hbm, gatet_hbm, ei_hbm, out_hbm,
              sidx, didx, rows, gbuft, h0_sp, agg_sp,
              gsem, isem, esem, ssem):
    cid = lax.axis_index("c")
    sid = lax.axis_index("s")
    wid = cid * NS + sid

    iota = lax.iota(jnp.int32, 16)
    rp = lax.shift_right_logical(iota, 3)
    cp = lax.bitwise_and(iota, 7)
    zero16 = jnp.zeros((16,), jnp.float32)

    def idx_issue(c):
        b3 = c % 3
        base2d = wid * (EPT // SUB) + c * NSUB
        return [
            pltpu.async_copy(ei_hbm.at[0, pl.ds(base2d, NSUB)], sidx[b3], isem[b3]),
            pltpu.async_copy(ei_hbm.at[1, pl.ds(base2d, NSUB)], didx[b3], isem[b3]),
        ]

    def gissue(c):
        b = c & 1
        b3 = c % 3
        base = wid * EPT + c * CH
        hs = [pltpu.async_copy(h0_sp.at[sidx[b3].at[j]],
                               rows[b].at[pl.ds(j * SUB, SUB)], gsem[b])
              for j in range(NSUB)]
        hs.append(pltpu.async_copy(gatet_hbm.at[:, pl.ds(base, CH)],
                                   gbuft[b], esem[b]))
        return hs

    ih = {0: idx_issue(0), 1: idx_issue(1)}

    @pl.when(sid == 0)
    def _stage():
        pltpu.sync_copy(h0_hbm, h0_sp)

    def _z(i, carry):
        plsc.store_scatter(rows[0], [rp + 2 * i, cp], zero16)
        return carry
    lax.fori_loop(0, (RPT + 2) // 2, _z, 0)
    pltpu.sync_copy(rows[0].at[pl.ds(0, RPT)],
                    agg_sp.at[pl.ds(sid * RPT, RPT)])
    plsc.subcore_barrier()

    for h in ih[0]:
        h.wait()
    gh = {0: gissue(0)}
    sh = {}
    for c in range(NCHUNK):
        b = c & 1
        b3 = c % 3
        if c + 1 < NCHUNK:
            for h in ih[c + 1]:
                h.wait()
            if c >= 1:
                for h in sh[c - 1]:
                    h.wait()
            gh[c + 1] = gissue(c + 1)
            if c + 2 < NCHUNK:
                ih[c + 2] = idx_issue(c + 2)
        for h in gh[c]:
            h.wait()

        def _m(i, carry2):
            r = rp + 2 * i
            m = plsc.load_gather(rows[b], [r, cp]) * plsc.load_gather(gbuft[b], [cp, r])
            plsc.store_scatter(rows[b], [r, cp], m)
            return carry2
        lax.fori_loop(0, CH // 2, _m, 0)

        sh[c] = [pltpu.async_copy(rows[b].at[pl.ds(j * SUB, SUB)],
                                  agg_sp.at[didx[b3].at[j]], ssem[b], add=True)
                 for j in range(NSUB)]
    for h in sh[NCHUNK - 2]:
        h.wait()
    for h in sh[NCHUNK - 1]:
        h.wait()

    plsc.subcore_barrier()

    @pl.when(sid == 0)
    def _out():
        pltpu.sync_copy(agg_sp, out_hbm.at[cid])


@functools.partial(
    pl.kernel,
    out_type=jax.ShapeDtypeStruct((NC, N, DH), jnp.float32),
    mesh=_MESH,
    compiler_params=_SC_PARAMS,
    scratch_types=[
        [pltpu.VMEM((NSUB, SUB), jnp.int32) for _ in range(3)],
        [pltpu.VMEM((NSUB, SUB), jnp.int32) for _ in range(3)],
        [pltpu.VMEM((CH, DH), jnp.float32) for _ in range(2)],
        [pltpu.VMEM((DH, CH), jnp.float32) for _ in range(2)],
        pltpu.VMEM_SHARED((N, DH), jnp.float32),
        pltpu.VMEM_SHARED((N, DH), jnp.float32),
        [pltpu.SemaphoreType.DMA for _ in range(2)],
        [pltpu.SemaphoreType.DMA for _ in range(3)],
        [pltpu.SemaphoreType.DMA for _ in range(2)],
        [pltpu.SemaphoreType.DMA for _ in range(2)],
    ],
)
def _sc_phase_a(h0_hbm, gatet_hbm, ei_hbm, out_hbm,
                sidx, didx, rows, gbuft, h0_sp, agg_sp, gsem, isem, esem,
                ssem):
    _sca_body(h0_hbm, gatet_hbm, ei_hbm, out_hbm,
              sidx, didx, rows, gbuft, h0_sp, agg_sp, gsem, isem, esem, ssem)


def _scc_body(h0_hbm, agg_hbm, w2_hbm, wp_hbm, bias_hbm, eat_hbm, ei_hbm,
              out_hbm,
              sidx, didx, rowsa, rowsb, eabuft, obuf, h0b, g0b, g1b,
              ab, bb, w2b, wpb, bbuf, a_sp, b_sp,
              gsem, isem, osem, psem, esem):
    cid = lax.axis_index("c")
    sid = lax.axis_index("s")
    wid = cid * NS + sid
    iota = lax.iota(jnp.int32, 16)
    kcols = [jnp.full((16,), k, jnp.int32) for k in range(DH)]

    def idx_issue(c):
        b3 = c % 3
        base2d = wid * (EPT // SUB) + c * NSUB
        return [
            pltpu.async_copy(ei_hbm.at[0, pl.ds(base2d, NSUB)], sidx[b3], isem[b3]),
            pltpu.async_copy(ei_hbm.at[1, pl.ds(base2d, NSUB)], didx[b3], isem[b3]),
        ]

    def gissue(c):
        b = c & 1
        b3 = c % 3
        base = wid * EPT + c * CH
        hs = []
        for j in range(NSUB):
            hs.append(pltpu.async_copy(
                a_sp.at[sidx[b3].at[j]], rowsa[b].at[pl.ds(j * SUB, SUB)], gsem[b]))
            hs.append(pltpu.async_copy(
                b_sp.at[didx[b3].at[j]], rowsb[b].at[pl.ds(j * SUB, SUB)], gsem[b]))
        hs.append(pltpu.async_copy(eat_hbm.at[:, pl.ds(base, CH)],
                                   eabuft[b], esem[b]))
        return hs

    ih = {0: idx_issue(0), 1: idx_issue(1)}

    nbase = sid * RPT
    ph = [pltpu.async_copy(h0_hbm.at[pl.ds(nbase, RPT)], h0b, psem),
          pltpu.async_copy(agg_hbm.at[0, pl.ds(nbase, RPT)], g0b, psem),
          pltpu.async_copy(agg_hbm.at[1, pl.ds(nbase, RPT)], g1b, psem),
          pltpu.async_copy(w2_hbm, w2b, psem),
          pltpu.async_copy(wp_hbm, wpb, psem),
          pltpu.async_copy(bias_hbm, bbuf, psem)]
    for h in ph:
        h.wait()
    w2v = [w2b[pl.ds(0, 16)], w2b[pl.ds(16, 16)],
           w2b[pl.ds(32, 16)], w2b[pl.ds(48, 16)]]
    w2s = [[jnp.broadcast_to(w2v[(DH * k + j) // 16][(DH * k + j) % 16], (16,))
            for j in range(DH)] for k in range(DH)]
    wpv = wpb[...]
    wpa = [jnp.broadcast_to(wpv[j], (16,)) for j in range(DH)]
    wpb_ = [jnp.broadcast_to(wpv[j + DH], (16,)) for j in range(DH)]
    one16 = jnp.full((16,), 1.0, jnp.float32)
    zero16 = jnp.zeros((16,), jnp.float32)

    def _nodegroup(start):
        r = start + iota
        aggk = []
        h0k = []
        for k in range(DH):
            h0k.append(plsc.load_gather(h0b, [r, kcols[k]]))
            aggk.append(plsc.load_gather(g0b, [r, kcols[k]])
                        + plsc.load_gather(g1b, [r, kcols[k]]))
        for j in range(DH):
            acc = h0k[j]
            for k in range(DH):
                acc = acc + aggk[k] * w2s[k][j]
            hp1 = jnp.maximum(acc, zero16) + one16
            plsc.store_scatter(ab, [r, kcols[j]], hp1 * wpa[j])
            plsc.store_scatter(bb, [r, kcols[j]], hp1 * wpb_[j])

    def _ng(i, carry):
        _nodegroup(16 * i)
        return carry
    lax.fori_loop(0, RPT // 16, _ng, 0)
    _nodegroup(RPT - 16)

    pltpu.sync_copy(ab, a_sp.at[pl.ds(nbase, RPT)])
    pltpu.sync_copy(bb, b_sp.at[pl.ds(nbase, RPT)])
    plsc.subcore_barrier()
    bias = bbuf[...]

    for h in ih[0]:
        h.wait()
    gh = {0: gissue(0)}
    oh = {}
    for c in range(NCHUNK):
        b = c & 1
        if c + 1 < NCHUNK:
            for h in ih[c + 1]:
                h.wait()
            gh[c + 1] = gissue(c + 1)
            if c + 2 < NCHUNK:
                ih[c + 2] = idx_issue(c + 2)
        for h in gh[c]:
            h.wait()
        if c >= 1:
            for h in oh[c - 1]:
                h.wait()

        def _group(start):
            r = start + iota
            acc = bias
            for k in range(DH):
                ga = plsc.load_gather(rowsa[b], [r, kcols[k]])
                gb = plsc.load_gather(rowsb[b], [r, kcols[k]])
                ea_k = eabuft[b][k, pl.ds(start, 16)]
                eb_k = eabuft[b][k + DH, pl.ds(start, 16)]
                acc = acc + ga * ea_k + gb * eb_k
            obuf[b][pl.ds(start, 16)] = acc

        def _g(i, carry2):
            _group(16 * i)
            return carry2
        lax.fori_loop(0, CH // 16, _g, 0)
        _group(CH - 16)
        base = wid * EPT + c * CH
        oh[c] = [pltpu.async_copy(obuf[b], out_hbm.at[pl.ds(base, CH)], osem[b])]
    for h in oh[NCHUNK - 1]:
        h.wait()


@functools.partial(
    pl.kernel,
    out_type=jax.ShapeDtypeStruct((E,), jnp.float32),
    mesh=_MESH,
    compiler_params=_SC_PARAMS,
    scratch_types=[
        [pltpu.VMEM((NSUB, SUB), jnp.int32) for _ in range(3)],
        [pltpu.VMEM((NSUB, SUB), jnp.int32) for _ in range(3)],
        [pltpu.VMEM((CH, DH), jnp.float32) for _ in range(2)],
        [pltpu.VMEM((CH, DH), jnp.float32) for _ in range(2)],
        [pltpu.VMEM((DE, CH), jnp.float32) for _ in range(2)],
        [pltpu.VMEM((CH,), jnp.float32) for _ in range(2)],
        pltpu.VMEM((RPT, DH), jnp.float32),
        pltpu.VMEM((RPT, DH), jnp.float32),
        pltpu.VMEM((RPT, DH), jnp.float32),
        pltpu.VMEM((RPT, DH), jnp.float32),
        pltpu.VMEM((RPT, DH), jnp.float32),
        pltpu.VMEM((DH * DH,), jnp.float32),
        pltpu.VMEM((DE,), jnp.float32),
        pltpu.VMEM((16,), jnp.float32),
        pltpu.VMEM_SHARED((N, DH), jnp.float32),
        pltpu.VMEM_SHARED((N, DH), jnp.float32),
        [pltpu.SemaphoreType.DMA for _ in range(2)],
        [pltpu.SemaphoreType.DMA for _ in range(3)],
        [pltpu.SemaphoreType.DMA for _ in range(2)],
        pltpu.SemaphoreType.DMA,
        [pltpu.SemaphoreType.DMA for _ in range(2)],
    ],
)
def _sc_phase_c(h0_hbm, agg_hbm, w2_hbm, wp_hbm, bias_hbm, eat_hbm, ei_hbm,
                out_hbm,
                sidx, didx, rowsa, rowsb, eabuft, obuf, h0b, g0b, g1b,
                ab, bb, w2b, wpb, bbuf, a_sp, b_sp, gsem, isem, osem, psem,
                esem):
    _scc_body(h0_hbm, agg_hbm, w2_hbm, wp_hbm, bias_hbm, eat_hbm, ei_hbm,
              out_hbm,
              sidx, didx, rowsa, rowsb, eabuft, obuf, h0b, g0b, g1b,
              ab, bb, w2b, wpb, bbuf, a_sp, b_sp, gsem, isem, osem, psem,
              esem)


def kernel(x, edge_index, edge_attr, W1, We, W2, Wp, bp):
    ei3 = edge_index.astype(jnp.int32).reshape(2, GSUB, SUB)

    h0, gatet = _tc1(x, W1, edge_attr, We)
    agg2 = _sc_phase_a(h0, gatet, ei3)

    wp_vec = Wp.reshape(DE)
    bias = jnp.broadcast_to(2.0 * bp[0], (16,))
    energy = _sc_phase_c(h0, agg2, W2.reshape(DH * DH), wp_vec, bias,
                         edge_attr.T, ei3)
    return energy.reshape(E, 1)


# speedup vs baseline: 1.0091x; 1.0091x over previous
import functools

import jax
import jax.numpy as jnp
from jax import lax
from jax.experimental import pallas as pl
from jax.experimental.pallas import tpu as pltpu
from jax.experimental.pallas import tpu_sc as plsc

N = 10000
E = 320000
DF = 128
DH = 8
DE = 16

NC, NS = 2, 16
NW = NC * NS
EPT = E // NW
SUB = 125
NSUB = 8
CH = SUB * NSUB
NCHUNK = EPT // CH
RPT = N // NS
GSUB = E // SUB

_MESH = plsc.VectorSubcoreMesh(core_axis_name="c", subcore_axis_name="s")
_SC_PARAMS = pltpu.CompilerParams(needs_layout_passes=False,
                                  use_tc_tiling_on_sc=False)


def _tc1_body(x_ref, w1_ref, eat_ref, wet_ref, h0_ref, gatet_ref):
    h0_ref[...] = jnp.maximum(
        jnp.dot(x_ref[...], w1_ref[...], preferred_element_type=jnp.float32),
        0.0)
    g = jnp.dot(wet_ref[...], eat_ref[...], preferred_element_type=jnp.float32)
    gatet_ref[...] = 1.0 / (1.0 + jnp.exp(-g))


def _tc1(x, W1, ea, We):
    grid = 10
    nb = N // grid
    eb = E // grid
    return pl.pallas_call(
        _tc1_body,
        grid=(grid,),
        in_specs=[
            pl.BlockSpec((nb, DF), lambda i: (i, 0)),
            pl.BlockSpec((DF, DH), lambda i: (0, 0)),
            pl.BlockSpec((DE, eb), lambda i: (0, i)),
            pl.BlockSpec((DH, DE), lambda i: (0, 0)),
        ],
        out_specs=[
            pl.BlockSpec((nb, DH), lambda i: (i, 0)),
            pl.BlockSpec((DH, eb), lambda i: (0, i)),
        ],
        out_shape=[
            jax.ShapeDtypeStruct((N, DH), jnp.float32),
            jax.ShapeDtypeStruct((DH, E), jnp.float32),
        ],
    )(x, W1, ea.T, We.T)


def _sca_body(h0_hbm, gatet_hbm, ei_hbm, out_hbm,
              sidx, didx, rows, gbuft, h0_sp, agg_sp,
              gsem, isem, esem, ssem):
    cid = lax.axis_index("c")
    sid = lax.axis_index("s")
    wid = cid * NS + sid

    iota = lax.iota(jnp.int32, 16)
    rp = lax.shift_right_logical(iota, 3)
    cp = lax.bitwise_and(iota, 7)
    zero16 = jnp.zeros((16,), jnp.float32)

    def idx_issue(c):
        b3 = c % 3
        base2d = wid * (EPT // SUB) + c * NSUB
        return [
            pltpu.async_copy(ei_hbm.at[0, pl.ds(base2d, NSUB)], sidx[b3], isem[b3]),
            pltpu.async_copy(ei_hbm.at[1, pl.ds(base2d, NSUB)], didx[b3], isem[b3]),
        ]

    def gissue(c):
        b = c & 1
        b3 = c % 3
        base = wid * EPT + c * CH
        hs = [pltpu.async_copy(h0_sp.at[sidx[b3].at[j]],
                               rows[b].at[pl.ds(j * SUB, SUB)], gsem[b])
              for j in range(NSUB)]
        hs.append(pltpu.async_copy(gatet_hbm.at[:, pl.ds(base, CH)],
                                   gbuft[b], esem[b]))
        return hs

    ih = {0: idx_issue(0), 1: idx_issue(1)}

    @pl.when(sid == 0)
    def _stage():
        pltpu.sync_copy(h0_hbm, h0_sp)

    def _z(i, carry):
        plsc.store_scatter(rows[0], [rp + 2 * i, cp], zero16)
        return carry
    lax.fori_loop(0, (RPT + 2) // 2, _z, 0)
    pltpu.sync_copy(rows[0].at[pl.ds(0, RPT)],
                    agg_sp.at[pl.ds(sid * RPT, RPT)])
    plsc.subcore_barrier()

    for h in ih[0]:
        h.wait()
    gh = {0: gissue(0)}
    sh = {}
    for c in range(NCHUNK):
        b = c & 1
        b3 = c % 3
        for h in gh[c]:
            h.wait()
        if c + 1 < NCHUNK:
            for h in ih[c + 1]:
                h.wait()
            if c >= 1:
                for h in sh[c - 1]:
                    h.wait()
            gh[c + 1] = gissue(c + 1)
            if c + 2 < NCHUNK:
                ih[c + 2] = idx_issue(c + 2)

        def _m(i, carry2):
            r = rp + 2 * i
            m = plsc.load_gather(rows[b], [r, cp]) * plsc.load_gather(gbuft[b], [cp, r])
            plsc.store_scatter(rows[b], [r, cp], m)
            return carry2
        lax.fori_loop(0, CH // 2, _m, 0)

        sh[c] = [pltpu.async_copy(rows[b].at[pl.ds(j * SUB, SUB)],
                                  agg_sp.at[didx[b3].at[j]], ssem[b], add=True)
                 for j in range(NSUB)]
    for h in sh[NCHUNK - 2]:
        h.wait()
    for h in sh[NCHUNK - 1]:
        h.wait()

    plsc.subcore_barrier()

    @pl.when(sid == 0)
    def _out():
        pltpu.sync_copy(agg_sp, out_hbm.at[cid])


@functools.partial(
    pl.kernel,
    out_type=jax.ShapeDtypeStruct((NC, N, DH), jnp.float32),
    mesh=_MESH,
    compiler_params=_SC_PARAMS,
    scratch_types=[
        [pltpu.VMEM((NSUB, SUB), jnp.int32) for _ in range(3)],
        [pltpu.VMEM((NSUB, SUB), jnp.int32) for _ in range(3)],
        [pltpu.VMEM((CH, DH), jnp.float32) for _ in range(2)],
        [pltpu.VMEM((DH, CH), jnp.float32) for _ in range(2)],
        pltpu.VMEM_SHARED((N, DH), jnp.float32),
        pltpu.VMEM_SHARED((N, DH), jnp.float32),
        [pltpu.SemaphoreType.DMA for _ in range(2)],
        [pltpu.SemaphoreType.DMA for _ in range(3)],
        [pltpu.SemaphoreType.DMA for _ in range(2)],
        [pltpu.SemaphoreType.DMA for _ in range(2)],
    ],
)
def _sc_phase_a(h0_hbm, gatet_hbm, ei_hbm, out_hbm,
                sidx, didx, rows, gbuft, h0_sp, agg_sp, gsem, isem, esem,
                ssem):
    _sca_body(h0_hbm, gatet_hbm, ei_hbm, out_hbm,
              sidx, didx, rows, gbuft, h0_sp, agg_sp, gsem, isem, esem, ssem)


def _scc_body(h0_hbm, agg_hbm, w2_hbm, wp_hbm, bias_hbm, eat_hbm, ei_hbm,
              out_hbm,
              sidx, didx, rowsa, rowsb, eabuft, obuf, h0b, g0b, g1b,
              ab, bb, w2b, wpb, bbuf, a_sp, b_sp,
              gsem, isem, osem, psem, esem):
    cid = lax.axis_index("c")
    sid = lax.axis_index("s")
    wid = cid * NS + sid
    iota = lax.iota(jnp.int32, 16)
    kcols = [jnp.full((16,), k, jnp.int32) for k in range(DH)]

    def idx_issue(c):
        b3 = c % 3
        base2d = wid * (EPT // SUB) + c * NSUB
        return [
            pltpu.async_copy(ei_hbm.at[0, pl.ds(base2d, NSUB)], sidx[b3], isem[b3]),
            pltpu.async_copy(ei_hbm.at[1, pl.ds(base2d, NSUB)], didx[b3], isem[b3]),
        ]

    def gissue(c):
        b = c & 1
        b3 = c % 3
        base = wid * EPT + c * CH
        hs = []
        for j in range(NSUB):
            hs.append(pltpu.async_copy(
                a_sp.at[sidx[b3].at[j]], rowsa[b].at[pl.ds(j * SUB, SUB)], gsem[b]))
            hs.append(pltpu.async_copy(
                b_sp.at[didx[b3].at[j]], rowsb[b].at[pl.ds(j * SUB, SUB)], gsem[b]))
        hs.append(pltpu.async_copy(eat_hbm.at[:, pl.ds(base, CH)],
                                   eabuft[b], esem[b]))
        return hs

    ih = {0: idx_issue(0), 1: idx_issue(1)}

    nbase = sid * RPT
    ph = [pltpu.async_copy(h0_hbm.at[pl.ds(nbase, RPT)], h0b, psem),
          pltpu.async_copy(agg_hbm.at[0, pl.ds(nbase, RPT)], g0b, psem),
          pltpu.async_copy(agg_hbm.at[1, pl.ds(nbase, RPT)], g1b, psem),
          pltpu.async_copy(w2_hbm, w2b, psem),
          pltpu.async_copy(wp_hbm, wpb, psem),
          pltpu.async_copy(bias_hbm, bbuf, psem)]
    for h in ph:
        h.wait()
    w2v = [w2b[pl.ds(0, 16)], w2b[pl.ds(16, 16)],
           w2b[pl.ds(32, 16)], w2b[pl.ds(48, 16)]]
    w2s = [[jnp.broadcast_to(w2v[(DH * k + j) // 16][(DH * k + j) % 16], (16,))
            for j in range(DH)] for k in range(DH)]
    wpv = wpb[...]
    wpa = [jnp.broadcast_to(wpv[j], (16,)) for j in range(DH)]
    wpb_ = [jnp.broadcast_to(wpv[j + DH], (16,)) for j in range(DH)]
    one16 = jnp.full((16,), 1.0, jnp.float32)
    zero16 = jnp.zeros((16,), jnp.float32)

    def _nodegroup(start):
        r = start + iota
        aggk = []
        h0k = []
        for k in range(DH):
            h0k.append(plsc.load_gather(h0b, [r, kcols[k]]))
            aggk.append(plsc.load_gather(g0b, [r, kcols[k]])
                        + plsc.load_gather(g1b, [r, kcols[k]]))
        for j in range(DH):
            acc = h0k[j]
            for k in range(DH):
                acc = acc + aggk[k] * w2s[k][j]
            hp1 = jnp.maximum(acc, zero16) + one16
            plsc.store_scatter(ab, [r, kcols[j]], hp1 * wpa[j])
            plsc.store_scatter(bb, [r, kcols[j]], hp1 * wpb_[j])

    def _ng(i, carry):
        _nodegroup(16 * i)
        return carry
    lax.fori_loop(0, RPT // 16, _ng, 0)
    _nodegroup(RPT - 16)

    pltpu.sync_copy(ab, a_sp.at[pl.ds(nbase, RPT)])
    pltpu.sync_copy(bb, b_sp.at[pl.ds(nbase, RPT)])
    plsc.subcore_barrier()
    bias = bbuf[...]

    for h in ih[0]:
        h.wait()
    gh = {0: gissue(0)}
    oh = {}
    for c in range(NCHUNK):
        b = c & 1
        for h in gh[c]:
            h.wait()
        if c + 1 < NCHUNK:
            for h in ih[c + 1]:
                h.wait()
            if c >= 1:
                for h in oh[c - 1]:
                    h.wait()
            gh[c + 1] = gissue(c + 1)
            if c + 2 < NCHUNK:
                ih[c + 2] = idx_issue(c + 2)

        def _group(start):
            r = start + iota
            acc = bias
            for k in range(DH):
                ga = plsc.load_gather(rowsa[b], [r, kcols[k]])
                gb = plsc.load_gather(rowsb[b], [r, kcols[k]])
                ea_k = eabuft[b][k, pl.ds(start, 16)]
                eb_k = eabuft[b][k + DH, pl.ds(start, 16)]
                acc = acc + ga * ea_k + gb * eb_k
            obuf[b][pl.ds(start, 16)] = acc

        def _g(i, carry2):
            _group(16 * i)
            return carry2
        lax.fori_loop(0, CH // 16, _g, 0)
        _group(CH - 16)
        base = wid * EPT + c * CH
        oh[c] = [pltpu.async_copy(obuf[b], out_hbm.at[pl.ds(base, CH)], osem[b])]
    for h in oh[NCHUNK - 2]:
        h.wait()
    for h in oh[NCHUNK - 1]:
        h.wait()


@functools.partial(
    pl.kernel,
    out_type=jax.ShapeDtypeStruct((E,), jnp.float32),
    mesh=_MESH,
    compiler_params=_SC_PARAMS,
    scratch_types=[
        [pltpu.VMEM((NSUB, SUB), jnp.int32) for _ in range(3)],
        [pltpu.VMEM((NSUB, SUB), jnp.int32) for _ in range(3)],
        [pltpu.VMEM((CH, DH), jnp.float32) for _ in range(2)],
        [pltpu.VMEM((CH, DH), jnp.float32) for _ in range(2)],
        [pltpu.VMEM((DE, CH), jnp.float32) for _ in range(2)],
        [pltpu.VMEM((CH,), jnp.float32) for _ in range(2)],
        pltpu.VMEM((RPT, DH), jnp.float32),
        pltpu.VMEM((RPT, DH), jnp.float32),
        pltpu.VMEM((RPT, DH), jnp.float32),
        pltpu.VMEM((RPT, DH), jnp.float32),
        pltpu.VMEM((RPT, DH), jnp.float32),
        pltpu.VMEM((DH * DH,), jnp.float32),
        pltpu.VMEM((DE,), jnp.float32),
        pltpu.VMEM((16,), jnp.float32),
        pltpu.VMEM_SHARED((N, DH), jnp.float32),
        pltpu.VMEM_SHARED((N, DH), jnp.float32),
        [pltpu.SemaphoreType.DMA for _ in range(2)],
        [pltpu.SemaphoreType.DMA for _ in range(3)],
        [pltpu.SemaphoreType.DMA for _ in range(2)],
        pltpu.SemaphoreType.DMA,
        [pltpu.SemaphoreType.DMA for _ in range(2)],
    ],
)
def _sc_phase_c(h0_hbm, agg_hbm, w2_hbm, wp_hbm, bias_hbm, eat_hbm, ei_hbm,
                out_hbm,
                sidx, didx, rowsa, rowsb, eabuft, obuf, h0b, g0b, g1b,
                ab, bb, w2b, wpb, bbuf, a_sp, b_sp, gsem, isem, osem, psem,
                esem):
    _scc_body(h0_hbm, agg_hbm, w2_hbm, wp_hbm, bias_hbm, eat_hbm, ei_hbm,
              out_hbm,
              sidx, didx, rowsa, rowsb, eabuft, obuf, h0b, g0b, g1b,
              ab, bb, w2b, wpb, bbuf, a_sp, b_sp, gsem, isem, osem, psem,
              esem)


def kernel(x, edge_index, edge_attr, W1, We, W2, Wp, bp):
    ei3 = edge_index.astype(jnp.int32).reshape(2, GSUB, SUB)

    h0, gatet = _tc1(x, W1, edge_attr, We)
    agg2 = _sc_phase_a(h0, gatet, ei3)

    wp_vec = Wp.reshape(DE)
    bias = jnp.broadcast_to(2.0 * bp[0], (16,))
    energy = _sc_phase_c(h0, agg2, W2.reshape(DH * DH), wp_vec, bias,
                         edge_attr.T, ei3)
    return energy.reshape(E, 1)
